# baseline (speedup 1.0000x reference)
import functools
import math

import jax
import jax.numpy as jnp
from jax import lax
from jax.experimental import pallas as pl
from jax.experimental.pallas import tpu as pltpu
from jax.experimental.pallas import tpu_sc as plsc

SMOOTH = 0.1
CONF = 1.0 - SMOOTH

_NC = 2
_NS = 16
_NW = _NC * _NS
_LANES = 16


def _sc_gather_body(v, per_w, flat_hbm, tgt_hbm, m_hbm, out_hbm,
                    tgt_v, idx_v, vals_v, m_v, acc_v, sem):
    wid = lax.axis_index("s") * _NC + lax.axis_index("c")
    base = wid * per_w
    pltpu.sync_copy(tgt_hbm.at[pl.ds(base, per_w)], tgt_v)
    pltpu.sync_copy(m_hbm.at[pl.ds(base, per_w)], m_v)
    for c in range(per_w // _LANES):
        t16 = tgt_v[pl.ds(c * _LANES, _LANES)]
        rows = lax.broadcasted_iota(jnp.int32, (_LANES,), 0)
        row0 = base + c * _LANES
        idx_v[pl.ds(c * _LANES, _LANES)] = (rows + row0) * v + t16
    pltpu.async_copy(flat_hbm.at[idx_v], vals_v, sem).wait()
    acc = jnp.zeros((_LANES,), jnp.float32)
    for c in range(per_w // _LANES):
        acc = acc + (vals_v[pl.ds(c * _LANES, _LANES)]
                     * m_v[pl.ds(c * _LANES, _LANES)])
    acc_v[...] = acc
    pltpu.sync_copy(acc_v, out_hbm.at[wid])


def _make_sc_gather(n_rows, v):
    per_w = n_rows // _NW
    mesh = plsc.VectorSubcoreMesh(core_axis_name="c", subcore_axis_name="s")
    return pl.kernel(
        functools.partial(_sc_gather_body, v, per_w),
        out_type=jax.ShapeDtypeStruct((_NW, _LANES), jnp.float32),
        mesh=mesh,
        scratch_types=[
            pltpu.VMEM((per_w,), jnp.int32),
            pltpu.VMEM((per_w,), jnp.int32),
            pltpu.VMEM((per_w,), jnp.float32),
            pltpu.VMEM((per_w,), jnp.float32),
            pltpu.VMEM((_LANES,), jnp.float32),
            pltpu.SemaphoreType.DMA,
        ],
    )


def _tc_body(nsteps_i, x_ref, m_ref, out_ref, acc_s, acc_m):
    i = pl.program_id(0)

    @pl.when(i == 0)
    def _init():
        acc_s[0, 0] = 0.0
        acc_m[0, 0] = 0.0

    x = x_ref[...]
    m = m_ref[...]
    rowsum = jnp.sum(x, axis=1, keepdims=True)
    acc_s[0, 0] += jnp.sum(rowsum * m)
    acc_m[0, 0] += jnp.sum(m)

    @pl.when(i == nsteps_i - 1)
    def _fin():
        out_ref[0, 0] = acc_s[0, 0]
        out_ref[0, 1] = acc_m[0, 0]


def _make_tc_sums(n_rows, v, block_r):
    ni = n_rows // block_r
    return pl.pallas_call(
        functools.partial(_tc_body, ni),
        grid=(ni,),
        in_specs=[
            pl.BlockSpec((block_r, v), lambda i: (i, 0)),
            pl.BlockSpec((block_r, 1), lambda i: (i, 0)),
        ],
        out_specs=pl.BlockSpec((1, 2), lambda i: (0, 0),
                               memory_space=pltpu.SMEM),
        out_shape=jax.ShapeDtypeStruct((1, 2), jnp.float32),
        scratch_shapes=[
            pltpu.SMEM((1, 1), jnp.float32),
            pltpu.SMEM((1, 1), jnp.float32),
        ],
    )


def kernel(input, target, mask):
    b, t, v = input.shape
    n = b * t
    x = input.reshape(n, v)
    tgt = target.reshape(n).astype(jnp.int32)
    m = mask.reshape(n)

    eps = SMOOTH / (v - 1)
    c_const = (v - 1) * eps * math.log(eps) + CONF * math.log(CONF)

    g_part = _make_sc_gather(n, v)(x.reshape(-1), tgt, m)
    sums = _make_tc_sums(n, v, 128)(x, m.reshape(n, 1))
    ts, mt = sums[0, 0], sums[0, 1]
    g_tot = jnp.sum(g_part)
    return (c_const * mt - eps * ts + (eps - CONF) * g_tot) / mt


# speedup vs baseline: 1.0039x; 1.0039x over previous
import functools
import math

import jax
import jax.numpy as jnp
from jax import lax
from jax.experimental import pallas as pl
from jax.experimental.pallas import tpu as pltpu
from jax.experimental.pallas import tpu_sc as plsc

SMOOTH = 0.1
CONF = 1.0 - SMOOTH

_NC = 2
_NS = 16
_NW = _NC * _NS
_LANES = 16

_ROWS_SC = 512


def _sc_body(v, gpw, rpw, flat_hbm, tgt_hbm, m_hbm, out_hbm,
             tgt_v, m_g, m_s, idx_v, gvals_v, buf0, buf1, acc_out,
             sem0, sem1, gsem):
    wid = lax.axis_index("s") * _NC + lax.axis_index("c")
    gbase = wid * gpw
    rbase = wid * rpw
    bufs = (buf0, buf1)
    sems = (sem0, sem1)

    h = pltpu.async_copy(flat_hbm.at[pl.ds(rbase * v, v)], buf0, sem0)

    pltpu.sync_copy(tgt_hbm.at[pl.ds(gbase, gpw)], tgt_v)
    pltpu.sync_copy(m_hbm.at[pl.ds(gbase, gpw)], m_g)
    pltpu.sync_copy(m_hbm.at[pl.ds(rbase, rpw)], m_s)
    for c in range(gpw // _LANES):
        t16 = tgt_v[pl.ds(c * _LANES, _LANES)]
        rows = lax.broadcasted_iota(jnp.int32, (_LANES,), 0)
        idx_v[pl.ds(c * _LANES, _LANES)] = (rows + gbase + c * _LANES) * v + t16
    pltpu.async_copy(flat_hbm.at[idx_v], gvals_v, gsem).wait()

    g16 = jnp.zeros((_LANES,), jnp.float32)
    ms16 = jnp.zeros((_LANES,), jnp.float32)
    for c in range(gpw // _LANES):
        g16 = g16 + (gvals_v[pl.ds(c * _LANES, _LANES)]
                     * m_g[pl.ds(c * _LANES, _LANES)])
        ms16 = ms16 + m_g[pl.ds(c * _LANES, _LANES)]

    nchunk = v // _LANES
    ts16 = jnp.zeros((_LANES,), jnp.float32)
    zero = jnp.zeros((_LANES,), jnp.float32)
    for r in range(rpw):
        if r + 1 < rpw:
            h_next = pltpu.async_copy(
                flat_hbm.at[pl.ds((rbase + r + 1) * v, v)],
                bufs[(r + 1) % 2], sems[(r + 1) % 2])
        h.wait()
        buf = bufs[r % 2]

        def _chunks(i, accs, buf=buf):
            return tuple(
                a + buf[pl.ds((i * 8 + k) * _LANES, _LANES)]
                for k, a in enumerate(accs))

        a0, a1, a2, a3, a4, a5, a6, a7 = lax.fori_loop(
            0, nchunk // 8, _chunks, (zero,) * 8)
        acc = ((a0 + a1) + (a2 + a3)) + ((a4 + a5) + (a6 + a7))
        mrow = m_s[pl.ds((r // _LANES) * _LANES, _LANES)][r % _LANES]
        ts16 = ts16 + acc * mrow
        if r + 1 < rpw:
            h = h_next

    acc_out[pl.ds(0, _LANES)] = ts16
    acc_out[pl.ds(_LANES, _LANES)] = g16
    acc_out[pl.ds(2 * _LANES, _LANES)] = ms16
    pltpu.sync_copy(acc_out, out_hbm.at[wid])


def _make_sc(n_rows, v):
    gpw = n_rows // _NW
    rpw = _ROWS_SC // _NW
    mesh = plsc.VectorSubcoreMesh(core_axis_name="c", subcore_axis_name="s")
    return pl.kernel(
        functools.partial(_sc_body, v, gpw, rpw),
        out_type=jax.ShapeDtypeStruct((_NW, 3 * _LANES), jnp.float32),
        mesh=mesh,
        scratch_types=[
            pltpu.VMEM((gpw,), jnp.int32),
            pltpu.VMEM((gpw,), jnp.float32),
            pltpu.VMEM((rpw,), jnp.float32),
            pltpu.VMEM((gpw,), jnp.int32),
            pltpu.VMEM((gpw,), jnp.float32),
            pltpu.VMEM((v,), jnp.float32),
            pltpu.VMEM((v,), jnp.float32),
            pltpu.VMEM((3 * _LANES,), jnp.float32),
            pltpu.SemaphoreType.DMA,
            pltpu.SemaphoreType.DMA,
            pltpu.SemaphoreType.DMA,
        ],
    )


def _tc_body(nsteps_i, x_ref, m_ref, out_ref, acc_s):
    i = pl.program_id(0)

    @pl.when(i == 0)
    def _init():
        acc_s[0, 0] = 0.0

    x = x_ref[...]
    m = m_ref[...]
    rowsum = jnp.sum(x, axis=1, keepdims=True)
    acc_s[0, 0] += jnp.sum(rowsum * m)

    @pl.when(i == nsteps_i - 1)
    def _fin():
        out_ref[0, 0] = acc_s[0, 0]


def _make_tc_sums(n_rows, v, block_r, row_off):
    ni = (n_rows - row_off) // block_r
    off_blocks = row_off // block_r
    return pl.pallas_call(
        functools.partial(_tc_body, ni),
        grid=(ni,),
        in_specs=[
            pl.BlockSpec((block_r, v), lambda i: (i + off_blocks, 0)),
            pl.BlockSpec((block_r, 1), lambda i: (i + off_blocks, 0)),
        ],
        out_specs=pl.BlockSpec((1, 1), lambda i: (0, 0),
                               memory_space=pltpu.SMEM),
        out_shape=jax.ShapeDtypeStruct((1, 1), jnp.float32),
        scratch_shapes=[
            pltpu.SMEM((1, 1), jnp.float32),
        ],
    )


def kernel(input, target, mask):
    b, t, v = input.shape
    n = b * t
    x = input.reshape(n, v)
    tgt = target.reshape(n).astype(jnp.int32)
    m = mask.reshape(n)

    eps = SMOOTH / (v - 1)
    c_const = (v - 1) * eps * math.log(eps) + CONF * math.log(CONF)

    sc_part = _make_sc(n, v)(x.reshape(-1), tgt, m)
    ts_tc = _make_tc_sums(n, v, 128, _ROWS_SC)(x, m.reshape(n, 1))[0, 0]

    sc3 = sc_part.reshape(_NW, 3, _LANES)
    ts = ts_tc + jnp.sum(sc3[:, 0, :])
    g_tot = jnp.sum(sc3[:, 1, :])
    mt = jnp.sum(sc3[:, 2, :])
    return (c_const * mt - eps * ts + (eps - CONF) * g_tot) / mt


# speedup vs baseline: 3.3169x; 3.3042x over previous
import functools
import math

import jax
import jax.numpy as jnp
from jax import lax
from jax.experimental import pallas as pl
from jax.experimental.pallas import tpu as pltpu

SMOOTH = 0.1
CONF = 1.0 - SMOOTH


def _tc_body(nsteps_i, c_const, eps, v,
             x_ref, m_ref, tgt_ref, out_ref, acc, acc_m):
    i = pl.program_id(0)

    @pl.when(i == 0)
    def _init():
        acc[0, 0] = 0.0
        acc_m[0, 0] = 0.0

    x = x_ref[...]
    m = m_ref[...]
    tgt = tgt_ref[...]
    br = x.shape[0]
    cols = lax.broadcasted_iota(jnp.int32, (br, v), 1)
    eq = cols == tgt
    rowsum = jnp.sum(x, axis=1, keepdims=True)
    grow = jnp.sum(jnp.where(eq, x, 0.0), axis=1, keepdims=True)
    acc[0, 0] += jnp.sum((-eps * rowsum + (eps - CONF) * grow) * m)
    acc_m[0, 0] += jnp.sum(m)

    @pl.when(i == nsteps_i - 1)
    def _fin():
        mt = acc_m[0, 0]
        out_ref[0, 0] = (c_const * mt + acc[0, 0]) / mt


def _make_tc_loss(n_rows, v, block_r):
    ni = n_rows // block_r
    eps = SMOOTH / (v - 1)
    c_const = (v - 1) * eps * math.log(eps) + CONF * math.log(CONF)
    return pl.pallas_call(
        functools.partial(_tc_body, ni, c_const, eps, v),
        grid=(ni,),
        in_specs=[
            pl.BlockSpec((block_r, v), lambda i: (i, 0)),
            pl.BlockSpec((block_r, 1), lambda i: (i, 0)),
            pl.BlockSpec((block_r, 1), lambda i: (i, 0)),
        ],
        out_specs=pl.BlockSpec((1, 1), lambda i: (0, 0),
                               memory_space=pltpu.SMEM),
        out_shape=jax.ShapeDtypeStruct((1, 1), jnp.float32),
        scratch_shapes=[
            pltpu.SMEM((1, 1), jnp.float32),
            pltpu.SMEM((1, 1), jnp.float32),
        ],
    )


def kernel(input, target, mask):
    b, t, v = input.shape
    n = b * t
    x = input.reshape(n, v)
    tgt = target.reshape(n, 1).astype(jnp.int32)
    m = mask.reshape(n, 1)
    return _make_tc_loss(n, v, 128)(x, m, tgt)[0, 0]
